# baseline (speedup 1.0000x reference)
import functools

import jax
import jax.numpy as jnp
from jax import lax
from jax.experimental import pallas as pl
from jax.experimental.pallas import tpu as pltpu
from jax.experimental.pallas import tpu_sc as plsc

N = 10000
D = 128
E = 320000
NC = 2
NS = 16
NW = NC * NS
K = 64
NCH = 160
NBUF = 4
NSS = 2
LKA = 2
UNR = 8
RV = 24
R_FAST = 8
R_SLOW = 6
NCHA = R_FAST * RV
E_FAST = NS * R_FAST * RV * K
E_SLOW = NS * R_SLOW * RV * K
N_PAD = 10240
RPT = N_PAD // NS

_MESH = plsc.VectorSubcoreMesh(core_axis_name="c", subcore_axis_name="s")


def _seg_body(hu, hi, sf, df, srb, drb, sr, dr, zfeat,
              of, orb, orr, src_v, dst_v, buf, acc, *sems):
  cid = lax.axis_index("c")
  sid = lax.axis_index("s")
  wid = sid * NC + cid
  row0 = sid * RPT
  gsems = sems[:NBUF]
  ssems = sems[NBUF:]

  for table, src_h, dst_h, out_h in (
      (hu, sf, df, of), (hi, srb, drb, orb), (hu, sr, dr, orr)):

    def gwait(j, b):
      pltpu.make_async_copy(table.at[src_v.at[j]], buf.at[b],
                            gsems[b]).wait()

    def swait(j, s):
      pltpu.make_async_copy(buf.at[j % NBUF], acc.at[dst_v.at[j]],
                            ssems[s]).wait()

    def visit(j, b, s, skip_swait=False, refill=True):
      gwait(j, b)
      if not skip_swait:
        swait(j - NSS, s)
      pltpu.async_copy(buf.at[b], acc.at[dst_v.at[j]], ssems[s], add=True)
      if refill:
        rb = (b + LKA) % NBUF
        pltpu.async_copy(table.at[src_v.at[j + LKA]], buf.at[rb], gsems[rb])

    pltpu.sync_copy(zfeat.at[pl.ds(row0, RPT)], acc.at[pl.ds(row0, RPT)])
    plsc.subcore_barrier()

    def round_body(rnd, carry):
      off = pl.multiple_of(rnd * RV, 8)
      pltpu.sync_copy(src_h.at[wid, pl.ds(off, RV)], src_v)
      pltpu.sync_copy(dst_h.at[wid, pl.ds(off, RV)], dst_v)
      for c in range(LKA):
        pltpu.async_copy(table.at[src_v.at[c]], buf.at[c], gsems[c])
      for j in range(UNR):
        visit(j, j % NBUF, j % NSS, skip_swait=j < NSS)

      def loop_body(i, c2):
        base = i * UNR
        for u in range(UNR):
          visit(base + u, u % NBUF, u % NSS)
        return c2

      lax.fori_loop(1, RV // UNR - 1, loop_body, 0)
      for j in range(RV - UNR, RV):
        visit(j, j % NBUF, j % NSS, refill=j + LKA < RV)
      for j in range(RV - NSS, RV):
        swait(j, j % NSS)
      return carry

    rounds_w = jnp.where(cid == 0, R_FAST, R_SLOW)
    lax.fori_loop(0, rounds_w, round_body, 0)

    plsc.subcore_barrier()
    pltpu.sync_copy(acc.at[pl.ds(row0, RPT)],
                    out_h.at[cid, pl.ds(row0, RPT)])


_SEG = pl.kernel(
    _seg_body,
    out_type=[jax.ShapeDtypeStruct((NC, N_PAD, D), jnp.float32)] * 3,
    mesh=_MESH,
    scratch_types=[
        pltpu.VMEM((RV, K), jnp.int32),
        pltpu.VMEM((RV, K), jnp.int32),
        pltpu.VMEM((NBUF, K, D), jnp.float32),
        pltpu.VMEM_SHARED((N_PAD, D), jnp.float32),
    ] + [pltpu.SemaphoreType.DMA] * (NBUF + NSS),
)


DEG_W = 128


def _deg_body(df, drb, dr, zdeg, ones_hbm,
              odf, odrb, odr, dst_v, ones_v, deg, ds):
  cid = lax.axis_index("c")
  sid = lax.axis_index("s")
  wid = sid * NC + cid
  row0 = sid * RPT
  pltpu.sync_copy(ones_hbm, ones_v)

  for dst_h, out_h in ((df, odf), (drb, odrb), (dr, odr)):
    pltpu.sync_copy(zdeg.at[pl.ds(row0, RPT)], deg.at[pl.ds(row0, RPT)])
    pltpu.sync_copy(dst_h.at[wid], dst_v)
    plsc.subcore_barrier()

    def fire(j, carry):
      pltpu.async_copy(ones_v, deg.at[dst_v.at[j]], ds, add=True)
      return carry

    lax.fori_loop(0, NCH, fire, 0)

    def drain(j, carry):
      pltpu.make_async_copy(ones_v, deg.at[dst_v.at[j]], ds).wait()
      return carry

    lax.fori_loop(0, NCH, drain, 0)
    plsc.subcore_barrier()
    pltpu.sync_copy(deg.at[pl.ds(row0, RPT)],
                    out_h.at[cid, pl.ds(row0, RPT)])


_DEG = pl.kernel(
    _deg_body,
    out_type=[jax.ShapeDtypeStruct((NC, N_PAD, DEG_W), jnp.float32)] * 3,
    mesh=_MESH,
    scratch_types=[
        pltpu.VMEM((NCH, K), jnp.int32),
        pltpu.VMEM((K, DEG_W), jnp.float32),
        pltpu.VMEM_SHARED((N_PAD, DEG_W), jnp.float32),
        pltpu.SemaphoreType.DMA,
    ],
)

BR = 1024


def _combine_body(final, sfr, srbr, srr, dfr, drbr, drr,
                  wf, bf, wrb, brb, wr, br, hu_o, hi_o):
  au = sfr[0] + sfr[1]
  arb = srbr[0] + srbr[1]
  ar = srr[0] + srr[1]
  degf = dfr[0, :, 0:1] + dfr[1, :, 0:1]
  degrb = drbr[0, :, 0:1] + drbr[1, :, 0:1]
  degr = drr[0, :, 0:1] + drr[1, :, 0:1]
  mu = (jnp.dot(au, wf[...], preferred_element_type=jnp.float32)
        / jnp.maximum(degf, 1.0) + bf[...])
  mu = mu + (jnp.dot(arb, wrb[...], preferred_element_type=jnp.float32)
             / jnp.maximum(degrb, 1.0) + brb[...])
  mi = (jnp.dot(ar, wr[...], preferred_element_type=jnp.float32)
        / jnp.maximum(degr, 1.0) + br[...])
  u = jnp.maximum(mu, 0.0)
  v = jnp.maximum(mi, 0.0)

  def l2(x):
    n = jnp.sqrt(jnp.sum(x * x, axis=1, keepdims=True))
    return x / jnp.maximum(n, 1e-12)

  u = l2(u)
  v = l2(v)
  if final:
    u = l2(u)
    v = l2(v)
  hu_o[...] = u
  hi_o[...] = v


def _combine(final):
  part = pl.BlockSpec((2, BR, D), lambda i: (0, i, 0))
  degs = pl.BlockSpec((2, BR, DEG_W), lambda i: (0, i, 0))
  wspec = pl.BlockSpec((D, D), lambda i: (0, 0))
  bspec = pl.BlockSpec((1, D), lambda i: (0, 0))
  return pl.pallas_call(
      functools.partial(_combine_body, final),
      grid=(N_PAD // BR,),
      in_specs=[part, part, part, degs, degs, degs,
                wspec, bspec, wspec, bspec, wspec, bspec],
      out_specs=[pl.BlockSpec((BR, D), lambda i: (i, 0))] * 2,
      out_shape=[jax.ShapeDtypeStruct((N_PAD, D), jnp.float32)] * 2,
  )


_COMBINE_MID = _combine(False)
_COMBINE_FIN = _combine(True)


_EF = (E * R_FAST // (R_FAST + R_SLOW)) // NS * NS


def _prep_edges_sym(edge):
  pad = NW * NCH * K - E
  src = jnp.concatenate([edge[0], jnp.zeros((pad,), jnp.int32)])
  dst = jnp.concatenate([edge[1], jnp.full((pad,), N_PAD - 1, jnp.int32)])
  return src.reshape(NW, NCH, K), dst.reshape(NW, NCH, K)


def _split_asym(x, fill):
  a = x[:_EF].reshape(NS, _EF // NS)
  a = jnp.concatenate(
      [a, jnp.full((NS, (E_FAST - _EF) // NS), fill, jnp.int32)], axis=1)
  a = a.reshape(NS, NCHA, K)
  b = x[_EF:].reshape(NS, (E - _EF) // NS)
  b = jnp.concatenate(
      [b, jnp.full((NS, (E_SLOW - (E - _EF)) // NS), fill, jnp.int32)],
      axis=1)
  b = b.reshape(NS, R_SLOW * RV, K)
  b = jnp.concatenate(
      [b, jnp.full((NS, (R_FAST - R_SLOW) * RV, K), fill, jnp.int32)],
      axis=1)
  return jnp.stack([a, b], axis=1).reshape(NW, NCHA, K)


def _prep_edges_asym(edge):
  return (_split_asym(edge[0], 0), _split_asym(edge[1], N_PAD - 1))


def _prep_table(h):
  return jnp.concatenate(
      [h, jnp.zeros((N_PAD - h.shape[0], D), jnp.float32)], axis=0)


def kernel(h_user, h_item, edge_follows, edge_rates, edge_ratedby,
           W0_follows, b0_follows, W0_rates, b0_rates, W0_ratedby, b0_ratedby,
           W1_follows, b1_follows, W1_rates, b1_rates, W1_ratedby, b1_ratedby):
  sf_s, sf_d = _prep_edges_asym(edge_follows)
  srb_s, srb_d = _prep_edges_asym(edge_ratedby)
  sr_s, sr_d = _prep_edges_asym(edge_rates)
  _, df_d = _prep_edges_sym(edge_follows)
  _, drb_d = _prep_edges_sym(edge_ratedby)
  _, dr_d = _prep_edges_sym(edge_rates)
  zfeat = jnp.zeros((N_PAD, D), jnp.float32)
  zdeg = jnp.zeros((N_PAD, DEG_W), jnp.float32)
  ones = jnp.ones((K, DEG_W), jnp.float32)
  tu = _prep_table(h_user)
  ti = _prep_table(h_item)

  Df, Drb, Dr = _DEG(df_d, drb_d, dr_d, zdeg, ones)
  Sf, Srb, Sr = _SEG(tu, ti, sf_s, sf_d, srb_s, srb_d, sr_s, sr_d, zfeat)
  hu1, hi1 = _COMBINE_MID(
      Sf, Srb, Sr, Df, Drb, Dr,
      W0_follows, b0_follows.reshape(1, D),
      W0_ratedby, b0_ratedby.reshape(1, D),
      W0_rates, b0_rates.reshape(1, D))
  Sf1, Srb1, Sr1 = _SEG(hu1, hi1, sf_s, sf_d, srb_s, srb_d, sr_s, sr_d, zfeat)
  hu2, hi2 = _COMBINE_FIN(
      Sf1, Srb1, Sr1, Df, Drb, Dr,
      W1_follows, b1_follows.reshape(1, D),
      W1_ratedby, b1_ratedby.reshape(1, D),
      W1_rates, b1_rates.reshape(1, D))
  return hu2[:N], hi2[:N]


# speedup vs baseline: 2.2827x; 2.2827x over previous
import functools

import jax
import jax.numpy as jnp
from jax import lax
from jax.experimental import pallas as pl
from jax.experimental.pallas import tpu as pltpu
from jax.experimental.pallas import tpu_sc as plsc

N = 10000
D = 128
E = 320000
NC = 2
NS = 16
NW = NC * NS
K = 64
NCH = 160
NBUF = 4
NSS = 2
LKA = 2
UNR = 8
RV = 40
R_FAST = 5
R_SLOW = 3
NCHA = R_FAST * RV
E_FAST = NS * R_FAST * RV * K
E_SLOW = NS * R_SLOW * RV * K
N_PAD = 10240
RPT = N_PAD // NS

_MESH = plsc.VectorSubcoreMesh(core_axis_name="c", subcore_axis_name="s")


def _seg_body(hu, hi, sf, df, srb, drb, sr, dr, zfeat,
              of, orb, orr, src_v, dst_v, buf, acc, *sems):
  cid = lax.axis_index("c")
  sid = lax.axis_index("s")
  wid = sid * NC + cid
  row0 = sid * RPT
  gsems = sems[:NBUF]
  ssems = sems[NBUF:]

  for table, src_h, dst_h, out_h in (
      (hu, sf, df, of), (hi, srb, drb, orb), (hu, sr, dr, orr)):

    def gwait(j, b):
      pltpu.make_async_copy(table.at[src_v.at[j]], buf.at[b],
                            gsems[b]).wait()

    def swait(j, s):
      pltpu.make_async_copy(buf.at[j % NBUF], acc.at[dst_v.at[j]],
                            ssems[s]).wait()

    def visit(j, b, s, skip_swait=False, refill=True):
      gwait(j, b)
      if not skip_swait:
        swait(j - NSS, s)
      pltpu.async_copy(buf.at[b], acc.at[dst_v.at[j]], ssems[s], add=True)
      if refill:
        rb = (b + LKA) % NBUF
        pltpu.async_copy(table.at[src_v.at[j + LKA]], buf.at[rb], gsems[rb])

    pltpu.sync_copy(zfeat.at[pl.ds(row0, RPT)], acc.at[pl.ds(row0, RPT)])
    plsc.subcore_barrier()

    def round_body(rnd, carry):
      off = pl.multiple_of(rnd * RV, 8)
      pltpu.sync_copy(src_h.at[wid, pl.ds(off, RV)], src_v)
      pltpu.sync_copy(dst_h.at[wid, pl.ds(off, RV)], dst_v)
      for c in range(LKA):
        pltpu.async_copy(table.at[src_v.at[c]], buf.at[c], gsems[c])
      for j in range(UNR):
        visit(j, j % NBUF, j % NSS, skip_swait=j < NSS)

      def loop_body(i, c2):
        base = i * UNR
        for u in range(UNR):
          visit(base + u, u % NBUF, u % NSS)
        return c2

      lax.fori_loop(1, RV // UNR - 1, loop_body, 0)
      for j in range(RV - UNR, RV):
        visit(j, j % NBUF, j % NSS, refill=j + LKA < RV)
      for j in range(RV - NSS, RV):
        swait(j, j % NSS)
      return carry

    rounds_w = jnp.where(cid == 0, R_FAST, R_SLOW)
    lax.fori_loop(0, rounds_w, round_body, 0)

    plsc.subcore_barrier()
    pltpu.sync_copy(acc.at[pl.ds(row0, RPT)],
                    out_h.at[cid, pl.ds(row0, RPT)])


_SEG = pl.kernel(
    _seg_body,
    out_type=[jax.ShapeDtypeStruct((NC, N_PAD, D), jnp.float32)] * 3,
    mesh=_MESH,
    scratch_types=[
        pltpu.VMEM((RV, K), jnp.int32),
        pltpu.VMEM((RV, K), jnp.int32),
        pltpu.VMEM((NBUF, K, D), jnp.float32),
        pltpu.VMEM_SHARED((N_PAD, D), jnp.float32),
    ] + [pltpu.SemaphoreType.DMA] * (NBUF + NSS),
)


DEG_W = 128


def _deg_body(df, drb, dr, zdeg, ones_hbm,
              odf, odrb, odr, dst_v, ones_v, deg, ds):
  cid = lax.axis_index("c")
  sid = lax.axis_index("s")
  wid = sid * NC + cid
  row0 = sid * RPT
  pltpu.sync_copy(ones_hbm, ones_v)

  for dst_h, out_h in ((df, odf), (drb, odrb), (dr, odr)):
    pltpu.sync_copy(zdeg.at[pl.ds(row0, RPT)], deg.at[pl.ds(row0, RPT)])
    pltpu.sync_copy(dst_h.at[wid], dst_v)
    plsc.subcore_barrier()

    def fire(j, carry):
      pltpu.async_copy(ones_v, deg.at[dst_v.at[j]], ds, add=True)
      return carry

    lax.fori_loop(0, NCH, fire, 0)

    def drain(j, carry):
      pltpu.make_async_copy(ones_v, deg.at[dst_v.at[j]], ds).wait()
      return carry

    lax.fori_loop(0, NCH, drain, 0)
    plsc.subcore_barrier()
    pltpu.sync_copy(deg.at[pl.ds(row0, RPT)],
                    out_h.at[cid, pl.ds(row0, RPT)])


_DEG = pl.kernel(
    _deg_body,
    out_type=[jax.ShapeDtypeStruct((NC, N_PAD, DEG_W), jnp.float32)] * 3,
    mesh=_MESH,
    scratch_types=[
        pltpu.VMEM((NCH, K), jnp.int32),
        pltpu.VMEM((K, DEG_W), jnp.float32),
        pltpu.VMEM_SHARED((N_PAD, DEG_W), jnp.float32),
        pltpu.SemaphoreType.DMA,
    ],
)

BR = 1024


def _combine_body(final, sfr, srbr, srr, dfr, drbr, drr,
                  wf, bf, wrb, brb, wr, br, hu_o, hi_o):
  au = sfr[0] + sfr[1]
  arb = srbr[0] + srbr[1]
  ar = srr[0] + srr[1]
  degf = dfr[0, :, 0:1] + dfr[1, :, 0:1]
  degrb = drbr[0, :, 0:1] + drbr[1, :, 0:1]
  degr = drr[0, :, 0:1] + drr[1, :, 0:1]
  mu = (jnp.dot(au, wf[...], preferred_element_type=jnp.float32)
        / jnp.maximum(degf, 1.0) + bf[...])
  mu = mu + (jnp.dot(arb, wrb[...], preferred_element_type=jnp.float32)
             / jnp.maximum(degrb, 1.0) + brb[...])
  mi = (jnp.dot(ar, wr[...], preferred_element_type=jnp.float32)
        / jnp.maximum(degr, 1.0) + br[...])
  u = jnp.maximum(mu, 0.0)
  v = jnp.maximum(mi, 0.0)

  def l2(x):
    n = jnp.sqrt(jnp.sum(x * x, axis=1, keepdims=True))
    return x / jnp.maximum(n, 1e-12)

  u = l2(u)
  v = l2(v)
  if final:
    u = l2(u)
    v = l2(v)
  hu_o[...] = u
  hi_o[...] = v


def _combine(final):
  part = pl.BlockSpec((2, BR, D), lambda i: (0, i, 0))
  degs = pl.BlockSpec((2, BR, DEG_W), lambda i: (0, i, 0))
  wspec = pl.BlockSpec((D, D), lambda i: (0, 0))
  bspec = pl.BlockSpec((1, D), lambda i: (0, 0))
  return pl.pallas_call(
      functools.partial(_combine_body, final),
      grid=(N_PAD // BR,),
      in_specs=[part, part, part, degs, degs, degs,
                wspec, bspec, wspec, bspec, wspec, bspec],
      out_specs=[pl.BlockSpec((BR, D), lambda i: (i, 0))] * 2,
      out_shape=[jax.ShapeDtypeStruct((N_PAD, D), jnp.float32)] * 2,
  )


_COMBINE_MID = _combine(False)
_COMBINE_FIN = _combine(True)


_EF = E * R_FAST // (R_FAST + R_SLOW)


def _prep_edges_sym(edge):
  pad = NW * NCH * K - E
  src = jnp.concatenate([edge[0], jnp.zeros((pad,), jnp.int32)])
  dst = jnp.concatenate([edge[1], jnp.full((pad,), N_PAD - 1, jnp.int32)])
  return src.reshape(NW, NCH, K), dst.reshape(NW, NCH, K)


def _split_asym(x, fill):
  a = x[:_EF].reshape(NS, _EF // NS)
  a = jnp.concatenate(
      [a, jnp.full((NS, (E_FAST - _EF) // NS), fill, jnp.int32)], axis=1)
  a = a.reshape(NS, NCHA, K)
  b = x[_EF:].reshape(NS, (E - _EF) // NS)
  b = jnp.concatenate(
      [b, jnp.full((NS, (E_SLOW - (E - _EF)) // NS), fill, jnp.int32)],
      axis=1)
  b = b.reshape(NS, R_SLOW * RV, K)
  b = jnp.concatenate(
      [b, jnp.full((NS, (R_FAST - R_SLOW) * RV, K), fill, jnp.int32)],
      axis=1)
  return jnp.stack([a, b], axis=1).reshape(NW, NCHA, K)


def _prep_edges_asym(edge):
  return (_split_asym(edge[0], 0), _split_asym(edge[1], N_PAD - 1))


def _prep_table(h):
  return jnp.concatenate(
      [h, jnp.zeros((N_PAD - h.shape[0], D), jnp.float32)], axis=0)


def kernel(h_user, h_item, edge_follows, edge_rates, edge_ratedby,
           W0_follows, b0_follows, W0_rates, b0_rates, W0_ratedby, b0_ratedby,
           W1_follows, b1_follows, W1_rates, b1_rates, W1_ratedby, b1_ratedby):
  sf_s, sf_d = _prep_edges_asym(edge_follows)
  srb_s, srb_d = _prep_edges_asym(edge_ratedby)
  sr_s, sr_d = _prep_edges_asym(edge_rates)
  _, df_d = _prep_edges_sym(edge_follows)
  _, drb_d = _prep_edges_sym(edge_ratedby)
  _, dr_d = _prep_edges_sym(edge_rates)
  zfeat = jnp.zeros((N_PAD, D), jnp.float32)
  zdeg = jnp.zeros((N_PAD, DEG_W), jnp.float32)
  ones = jnp.ones((K, DEG_W), jnp.float32)
  tu = _prep_table(h_user)
  ti = _prep_table(h_item)

  Df, Drb, Dr = _DEG(df_d, drb_d, dr_d, zdeg, ones)
  Sf, Srb, Sr = _SEG(tu, ti, sf_s, sf_d, srb_s, srb_d, sr_s, sr_d, zfeat)
  hu1, hi1 = _COMBINE_MID(
      Sf, Srb, Sr, Df, Drb, Dr,
      W0_follows, b0_follows.reshape(1, D),
      W0_ratedby, b0_ratedby.reshape(1, D),
      W0_rates, b0_rates.reshape(1, D))
  Sf1, Srb1, Sr1 = _SEG(hu1, hi1, sf_s, sf_d, srb_s, srb_d, sr_s, sr_d, zfeat)
  hu2, hi2 = _COMBINE_FIN(
      Sf1, Srb1, Sr1, Df, Drb, Dr,
      W1_follows, b1_follows.reshape(1, D),
      W1_ratedby, b1_ratedby.reshape(1, D),
      W1_rates, b1_rates.reshape(1, D))
  return hu2[:N], hi2[:N]
